# baseline (speedup 1.0000x reference)
import functools

import jax
import jax.numpy as jnp
import numpy as np
from jax import lax
from jax.experimental import pallas as pl
from jax.experimental.pallas import tpu as pltpu
from jax.experimental.pallas import tpu_sc as plsc

_NC = 2
_NS = 16
_NW = _NC * _NS
_CHUNK = 128
_NBUF = 2
_LANES = 16
_OFFS = (0, 16, 32, 48, 64, 80, 84)
_RSQRT_MAGIC = np.int32(0x5F3759DF)


def _make_kernel(total, pair_rows, dim, seq_len):
    per_w = total // _NW
    nch = per_w // _CHUNK
    mesh = plsc.VectorSubcoreMesh(core_axis_name="c", subcore_axis_name="s")

    @functools.partial(
        pl.kernel,
        out_type=jax.ShapeDtypeStruct((total, dim), jnp.float32),
        mesh=mesh,
        scratch_types=[
            pltpu.VMEM((nch, _CHUNK), jnp.int32),
            pltpu.VMEM((nch, _CHUNK + _LANES), jnp.int32),
            pltpu.VMEM((seq_len, dim), jnp.float32),
            pltpu.VMEM((dim,), jnp.float32),
            pltpu.VMEM((dim,), jnp.float32),
            pltpu.VMEM((dim,), jnp.float32),
            pltpu.VMEM((_CHUNK, 2 * dim), jnp.float32),
            pltpu.VMEM((_CHUNK, 2 * dim), jnp.float32),
            pltpu.VMEM((_CHUNK, dim), jnp.float32),
            pltpu.VMEM((_CHUNK, dim), jnp.float32),
            pltpu.SemaphoreType.DMA,
            pltpu.SemaphoreType.DMA,
            pltpu.SemaphoreType.DMA,
            pltpu.SemaphoreType.DMA,
        ],
        compiler_params=pltpu.CompilerParams(use_tc_tiling_on_sc=False),
    )
    def ln_embed(pidx_hbm, aux_hbm, table_hbm, last_hbm, pos_hbm, gamma_hbm,
                 beta_hbm, out_hbm, idx_v, aux_v, pos_v, g_v, b_v, lr_v,
                 in0, in1, o0, o1, gs0, gs1, ws0, ws1):
        wid = lax.axis_index("s") * _NC + lax.axis_index("c")
        ins = (in0, in1)
        outs = (o0, o1)
        gsem = (gs0, gs1)
        wsem = (ws0, ws1)

        pltpu.sync_copy(pidx_hbm.at[wid], idx_v)
        pltpu.sync_copy(aux_hbm.at[wid], aux_v)
        pltpu.sync_copy(pos_hbm, pos_v)
        pltpu.sync_copy(gamma_hbm, g_v)
        pltpu.sync_copy(beta_hbm, b_v)
        pltpu.sync_copy(last_hbm, lr_v)

        def gather_copy(k, b):
            return pltpu.make_async_copy(
                table_hbm.at[idx_v.at[k]], ins[b], gsem[b])

        def write_copy(k, b):
            base = wid * per_w + k * _CHUNK
            return pltpu.make_async_copy(
                outs[b], out_hbm.at[pl.ds(base, _CHUNK)], wsem[b])

        lane = lax.iota(jnp.int32, _LANES)
        tail_mask = lane >= (_LANES - dim % _LANES)
        perms = [lane ^ s for s in (8, 4, 2, 1)]

        def lane_sum(x):
            for p in perms:
                x = x + x.at[p].get(mode="promise_in_bounds")
            return x

        gammas = [g_v[pl.ds(o, _LANES)] for o in _OFFS]
        betas = [b_v[pl.ds(o, _LANES)] for o in _OFFS]
        lasts = [lr_v[pl.ds(o, _LANES)] for o in _OFFS]
        inv_dim = np.float32(1.0 / dim)

        def do_row(k, in_ref, out_ref, i, l):
            aux = aux_v[k, pl.ds(i, _LANES)][0]
            cb = (aux & 1) * dim
            il = aux >= 2
            xs = [jnp.where(il, lasts[c], in_ref[i, pl.ds(cb + o, _LANES)])
                  + pos_v[l, pl.ds(o, _LANES)]
                  for c, o in enumerate(_OFFS)]
            xm = jnp.where(tail_mask, xs[6], np.float32(0.0))
            ts = ((xs[0] + xs[1]) + (xs[2] + xs[3])) + ((xs[4] + xs[5]) + xm)
            sq = [x * x for x in xs[:6]] + [xm * xm]
            tq = ((sq[0] + sq[1]) + (sq[2] + sq[3])) + ((sq[4] + sq[5]) + sq[6])
            mv = lane_sum(ts) * inv_dim
            ex2 = lane_sum(tq) * inv_dim
            vx = ex2 - mv * mv + np.float32(1e-6)
            yi = _RSQRT_MAGIC - lax.shift_right_logical(
                lax.bitcast_convert_type(vx, jnp.int32), 1)
            y = lax.bitcast_convert_type(yi, jnp.float32)
            for _ in range(3):
                y = y * (np.float32(1.5) - np.float32(0.5) * vx * y * y)
            for c, o in enumerate(_OFFS):
                out_ref[i, pl.ds(o, _LANES)] = (
                    ((xs[c] - mv) * y) * gammas[c] + betas[c])

        def do_chunk(k, in_ref, out_ref):
            lbase = lax.rem(k * _CHUNK, seq_len)

            def row_body(i, l):
                do_row(k, in_ref, out_ref, i, l)
                l2 = l + 1
                return jnp.where(l2 == seq_len, 0, l2)

            lax.fori_loop(0, _CHUNK, row_body, lbase)

        for b in range(_NBUF):
            gather_copy(b, b).start()

        def outer(j, carry):
            for b in range(_NBUF):
                k = j * _NBUF + b
                gather_copy(k, b).wait()

                @pl.when(j > 0)
                def _():
                    write_copy(k - _NBUF, b).wait()

                do_chunk(k, ins[b], outs[b])
                write_copy(k, b).start()

                @pl.when(j < nch // _NBUF - 1)
                def _():
                    gather_copy(k + _NBUF, b).start()
            return carry

        lax.fori_loop(0, nch // _NBUF, outer, np.int32(0))
        for b in range(_NBUF):
            write_copy(nch - _NBUF + b, b).wait()

    return ln_embed


def kernel(sen, table, pos_emb, gamma, beta):
    b, l = sen.shape
    vocab, dim = table.shape
    total = b * l
    per_w = total // _NW
    nch = per_w // _CHUNK
    pair_rows = (vocab - 1) // 2
    tab2 = table[: 2 * pair_rows].reshape(pair_rows, 2 * dim)
    last_row = table[vocab - 1]
    idxf = sen.reshape(-1).astype(jnp.int32)
    is_last = idxf == (vocab - 1)
    pidx = jnp.where(is_last, 0, lax.shift_right_logical(idxf, 1))
    aux = (idxf & 1) + 2 * is_last.astype(jnp.int32)
    pidx3 = pidx.reshape(_NW, nch, _CHUNK)
    aux3 = jnp.pad(aux.reshape(_NW, nch, _CHUNK), ((0, 0), (0, 0), (0, _LANES)))
    fn = _make_kernel(total, pair_rows, dim, l)
    out = fn(pidx3, aux3, tab2, last_row, pos_emb[:l], gamma, beta)
    return out.reshape(b, l, dim)
